# Initial kernel scaffold; baseline (speedup 1.0000x reference)
import jax
import jax.numpy as jnp
from jax.experimental import pallas as pl


def kernel(x, rand_projs, CB):
    raise NotImplementedError("write your pallas kernel here")


# speedup vs baseline: 1.3891x; 1.3891x over previous
import jax
import jax.numpy as jnp
from jax.experimental import pallas as pl
from jax.experimental.pallas import tpu as pltpu

_B, _T, _DIM = 16, 576, 256
_H, _C, _K = 2, 1024, 64


def _rpq_kernel(x_ref, rp_ref, cbt_ref, dist_ref, idx_ref):
    x = x_ref[0]
    mean = jnp.mean(x, axis=-1, keepdims=True)
    d = x - mean
    var = jnp.mean(d * d, axis=-1, keepdims=True)
    xn = d / jnp.sqrt(var + 1e-5)
    xp = jnp.dot(xn, rp_ref[0], preferred_element_type=jnp.float32)
    cbt = cbt_ref[0]
    dot = jnp.dot(xp, cbt, preferred_element_type=jnp.float32)
    na = jnp.sqrt(jnp.sum(cbt * cbt, axis=0, keepdims=True))
    nb = jnp.sqrt(jnp.sum(xp * xp, axis=-1, keepdims=True))
    dist = dot / jnp.maximum(na * nb, 1e-8)
    dist_ref[0, 0] = dist
    idx_ref[0, 0] = jnp.argmax(dist, axis=-1).astype(jnp.int32)[None, :]


def kernel(x, rand_projs, CB):
    CBt = jnp.transpose(CB, (0, 2, 1))
    dist, idx = pl.pallas_call(
        _rpq_kernel,
        grid=(_H, _B),
        in_specs=[
            pl.BlockSpec((1, _T, _DIM), lambda h, b: (b, 0, 0)),
            pl.BlockSpec((1, _DIM, _K), lambda h, b: (h, 0, 0)),
            pl.BlockSpec((1, _K, _C), lambda h, b: (h, 0, 0)),
        ],
        out_specs=[
            pl.BlockSpec((1, 1, _T, _C), lambda h, b: (h, b, 0, 0)),
            pl.BlockSpec((1, 1, 1, _T), lambda h, b: (h, b, 0, 0)),
        ],
        out_shape=[
            jax.ShapeDtypeStruct((_H, _B, _T, _C), jnp.float32),
            jax.ShapeDtypeStruct((_H, _B, 1, _T), jnp.int32),
        ],
    )(x, rand_projs, CBt)
    indices = jnp.transpose(idx.reshape(_H, _B, _T), (1, 2, 0))
    return (indices, dist)


# speedup vs baseline: 1.7300x; 1.2454x over previous
import jax
import jax.numpy as jnp
from jax.experimental import pallas as pl
from jax.experimental.pallas import tpu as pltpu

_B, _T, _DIM = 16, 576, 256
_H, _C, _K = 2, 1024, 64


def _rpq_kernel(x_ref, rp_ref, cbt_ref, dist_ref, idx_ref):
    x = x_ref[0]
    mean = jnp.mean(x, axis=-1, keepdims=True)
    d = x - mean
    var = jnp.mean(d * d, axis=-1, keepdims=True)
    xn = d * jax.lax.rsqrt(var + 1e-5)
    for h in range(_H):
        xp = jnp.dot(xn, rp_ref[h], preferred_element_type=jnp.float32)
        cbt = cbt_ref[h]
        rna = jax.lax.rsqrt(jnp.sum(cbt * cbt, axis=0, keepdims=True))
        rnb = 1.0 / jnp.maximum(
            jnp.sqrt(jnp.sum(xp * xp, axis=-1, keepdims=True)), 1e-8)
        dist = jnp.dot(xp * rnb, cbt * rna, preferred_element_type=jnp.float32)
        dist_ref[h, 0] = dist
        idx_ref[h, 0] = jnp.argmax(dist, axis=-1).astype(jnp.int32)[None, :]


def kernel(x, rand_projs, CB):
    CBt = jnp.transpose(CB, (0, 2, 1))
    dist, idx = pl.pallas_call(
        _rpq_kernel,
        grid=(_B,),
        in_specs=[
            pl.BlockSpec((1, _T, _DIM), lambda b: (b, 0, 0)),
            pl.BlockSpec((_H, _DIM, _K), lambda b: (0, 0, 0)),
            pl.BlockSpec((_H, _K, _C), lambda b: (0, 0, 0)),
        ],
        out_specs=[
            pl.BlockSpec((_H, 1, _T, _C), lambda b: (0, b, 0, 0)),
            pl.BlockSpec((_H, 1, 1, _T), lambda b: (0, b, 0, 0)),
        ],
        out_shape=[
            jax.ShapeDtypeStruct((_H, _B, _T, _C), jnp.float32),
            jax.ShapeDtypeStruct((_H, _B, 1, _T), jnp.int32),
        ],
    )(x, rand_projs, CBt)
    indices = jnp.transpose(idx.reshape(_H, _B, _T), (1, 2, 0))
    return (indices, dist)
